# baseline (speedup 1.0000x reference)
import functools

import jax
import jax.numpy as jnp
from jax import lax
from jax.experimental import pallas as pl
from jax.experimental.pallas import tpu as pltpu
from jax.experimental.pallas import tpu_sc as plsc

NUM_USERS = 1000000
EMBED_DIM = 64
BATCH = 16384

_info = plsc.get_sparse_core_info()
_NC = _info.num_cores
_NS = _info.num_subcores
_NW = _NC * _NS

_CW = 128
_TAIL = (NUM_USERS // 128) * 128
_NCH = _TAIL // _CW
_CPW = _NCH // _NW
_XTRA = _NCH - _CPW * _NW
_NBUF = 4
_ORING = 32
_CSH = 7


@functools.partial(
    pl.kernel,
    mesh=plsc.VectorSubcoreMesh(core_axis_name="c", subcore_axis_name="s"),
    out_type=jax.ShapeDtypeStruct((BATCH * EMBED_DIM,), jnp.float32),
    scratch_types=[
        pltpu.VMEM((BATCH,), jnp.int32),
        pltpu.VMEM((BATCH + 16,), jnp.int32),
        pltpu.VMEM((BATCH + 16,), jnp.int32),
        pltpu.VMEM((BATCH + 16,), jnp.int32),
        pltpu.VMEM((BATCH + 16,), jnp.int32),
        [pltpu.VMEM((EMBED_DIM, _CW), jnp.float32) for _ in range(_NBUF)],
        pltpu.VMEM((EMBED_DIM, EMBED_DIM), jnp.float32),
        pltpu.VMEM((_ORING * EMBED_DIM,), jnp.float32),
        pltpu.SMEM((256,), jnp.int32),
        pltpu.SMEM((256,), jnp.int32),
        pltpu.SMEM((256,), jnp.int32),
        pltpu.SMEM((8,), jnp.int32),
        [pltpu.SemaphoreType.DMA for _ in range(_NBUF)],
        pltpu.SemaphoreType.DMA,
    ],
    compiler_params=pltpu.CompilerParams(needs_layout_passes=False),
)
def _sc_gather(idx_hbm, tt_hbm, tail_hbm, out_hbm,
               idx_all, pu, pg, su, sg, bufs, tail_v, oring,
               counts, starts, cursor, scal,
               gsems, osem):
    wid = lax.axis_index("s") * _NC + lax.axis_index("c")
    base_c = wid * _CPW + jnp.minimum(wid, _XTRA)
    n_c = _CPW + (wid < _XTRA).astype(jnp.int32)

    lanes = lax.iota(jnp.int32, 16)
    z16 = jnp.zeros((16,), jnp.int32)
    lane0 = lanes == 0

    pltpu.sync_copy(idx_hbm, idx_all)
    pltpu.sync_copy(tail_hbm, tail_v)

    def p1_body(v, off):
        iv = idx_all[pl.ds(v * 16, 16)]
        cu = jnp.minimum(iv, _TAIL - 1)
        c = cu >> _CSH
        m = (c >= base_c) & (c < base_c + n_c)
        mi = m.astype(jnp.int32)
        cnt = jnp.sum(mi)

        @pl.when(cnt > 0)
        def _():
            pos = jnp.minimum(off + plsc.cumsum(mi) - 1, BATCH - 1)
            plsc.store_scatter(pu, [pos], iv, mask=m)
            plsc.store_scatter(pg, [pos], v * 16 + lanes, mask=m)

        return off + cnt

    off = lax.fori_loop(0, BATCH // 16, p1_body, 0)

    def zinit(j, _):
        counts[j] = 0
        return 0

    lax.fori_loop(0, 256, zinit, 0)

    def p2_body(p, _):
        u = pu[pl.ds(p, 16)][0]
        c_l = (jnp.minimum(u, _TAIL - 1) >> _CSH) - base_c
        counts[c_l] = counts[c_l] + 1
        return 0

    lax.fori_loop(0, off, p2_body, 0)

    def p3_body(j, run):
        starts[j] = run
        cursor[j] = run
        return run + counts[j]

    lax.fori_loop(0, 256, p3_body, 0)

    def p4_body(p, _):
        u = pu[pl.ds(p, 16)][0]
        g = pg[pl.ds(p, 16)][0]
        c_l = (jnp.minimum(u, _TAIL - 1) >> _CSH) - base_c
        pos = cursor[c_l]
        cursor[c_l] = pos + 1
        plsc.store_scatter(su, [z16 + pos], z16 + u, mask=lane0)
        plsc.store_scatter(sg, [z16 + pos], z16 + g, mask=lane0)
        return 0

    lax.fori_loop(0, off, p4_body, 0)

    scal[0] = 0

    def hit(c):
        return (c < n_c) & (counts[c] > 0)

    def issue(c, s):
        start = pl.multiple_of((base_c + c) * _CW, 128)
        pltpu.async_copy(tt_hbm.at[:, pl.ds(start, _CW)], bufs[s], gsems[s])

    def drain(s):
        pltpu.make_async_copy(
            tt_hbm.at[:, pl.ds(0, _CW)], bufs[s], gsems[s]
        ).wait()

    def extract_chunk(c, s):
        cbase = (base_c + c) * _CW
        s0 = starts[c]
        cnt = counts[c]

        def ubody(p, _):
            u = su[pl.ds(p, 16)][0]
            g = sg[pl.ds(p, 16)][0]
            q = scal[0]
            scal[0] = q + 1
            slot = q & (_ORING - 1)
            obase = pl.multiple_of(slot * EMBED_DIM, EMBED_DIM)

            @pl.when(q >= _ORING)
            def _():
                pltpu.make_async_copy(
                    out_hbm.at[pl.ds(0, EMBED_DIM)],
                    oring.at[pl.ds(0, EMBED_DIM)],
                    osem,
                ).wait()

            lane_v = z16 + jnp.minimum(u - cbase, _CW - 1)

            @pl.when(u < _TAIL)
            def _():
                for k in range(EMBED_DIM // 16):
                    dvec = lanes + (k * 16)
                    g1 = plsc.load_gather(bufs[s], [dvec, lane_v])
                    oring[pl.ds(obase + k * 16, 16)] = g1

            @pl.when(u >= _TAIL)
            def _():
                tl_v = z16 + jnp.minimum(
                    jnp.maximum(u - _TAIL, 0), EMBED_DIM - 1
                )
                for k in range(EMBED_DIM // 16):
                    dvec = lanes + (k * 16)
                    g2 = plsc.load_gather(tail_v, [tl_v, dvec])
                    oring[pl.ds(obase + k * 16, 16)] = g2
            pltpu.async_copy(
                oring.at[pl.ds(obase, EMBED_DIM)],
                out_hbm.at[pl.ds(g * EMBED_DIM, EMBED_DIM)],
                osem,
            )
            return 0

        lax.fori_loop(s0, s0 + cnt, ubody, 0)

    for s in range(_NBUF):
        @pl.when(hit(s))
        def _(s=s):
            issue(s, s)

    def ring_body(t, _):
        for s in range(_NBUF):
            c = t * _NBUF + s

            @pl.when(hit(c))
            def _(c=c, s=s):
                drain(s)
                extract_chunk(c, s)

            @pl.when(hit(c + _NBUF))
            def _(c=c, s=s):
                issue(c + _NBUF, s)
        return 0

    lax.fori_loop(0, (_CPW + _XTRA + _NBUF - 1) // _NBUF, ring_body, 0)

    def odrain(r, _):
        pltpu.make_async_copy(
            out_hbm.at[pl.ds(0, EMBED_DIM)],
            oring.at[pl.ds(0, EMBED_DIM)],
            osem,
        ).wait()
        return 0

    lax.fori_loop(0, jnp.minimum(off, _ORING), odrain, 0)


def kernel(user_inputs, table):
    tt = jnp.transpose(table)
    tail = table[_TAIL:, :]
    out_flat = _sc_gather(user_inputs, tt, tail)
    return jnp.reshape(out_flat, (BATCH, EMBED_DIM))


# speedup vs baseline: 1.0818x; 1.0818x over previous
import functools

import jax
import jax.numpy as jnp
from jax import lax
from jax.experimental import pallas as pl
from jax.experimental.pallas import tpu as pltpu
from jax.experimental.pallas import tpu_sc as plsc

NUM_USERS = 1000000
EMBED_DIM = 64
BATCH = 16384

_info = plsc.get_sparse_core_info()
_NC = _info.num_cores
_NS = _info.num_subcores
_NW = _NC * _NS

_CW = 128
_TAIL = (NUM_USERS // 128) * 128
_NCH = _TAIL // _CW
_CPW = _NCH // _NW
_XTRA = _NCH - _CPW * _NW
_NBUF = 4
_ORING = 32
_CSH = 7


@functools.partial(
    pl.kernel,
    mesh=plsc.VectorSubcoreMesh(core_axis_name="c", subcore_axis_name="s"),
    out_type=jax.ShapeDtypeStruct((BATCH * EMBED_DIM,), jnp.float32),
    scratch_types=[
        pltpu.VMEM((BATCH,), jnp.int32),
        pltpu.VMEM((BATCH + 16,), jnp.int32),
        pltpu.VMEM((BATCH + 16,), jnp.int32),
        pltpu.VMEM((BATCH + 16,), jnp.int32),
        pltpu.VMEM((BATCH + 16,), jnp.int32),
        [pltpu.VMEM((EMBED_DIM, _CW), jnp.float32) for _ in range(_NBUF)],
        pltpu.VMEM((EMBED_DIM, EMBED_DIM), jnp.float32),
        pltpu.VMEM((_ORING * EMBED_DIM,), jnp.float32),
        pltpu.VMEM((272,), jnp.int32),
        pltpu.VMEM((272,), jnp.int32),
        pltpu.VMEM((272,), jnp.int32),
        pltpu.SMEM((8,), jnp.int32),
        [pltpu.SemaphoreType.DMA for _ in range(_NBUF)],
        pltpu.SemaphoreType.DMA,
    ],
    compiler_params=pltpu.CompilerParams(needs_layout_passes=False),
)
def _sc_gather(idx_hbm, tt_hbm, tail_hbm, out_hbm,
               idx_all, pu, pg, su, sg, bufs, tail_v, oring,
               counts, starts, cursor, scal,
               gsems, osem):
    wid = lax.axis_index("s") * _NC + lax.axis_index("c")
    base_c = wid * _CPW + jnp.minimum(wid, _XTRA)
    n_c = _CPW + (wid < _XTRA).astype(jnp.int32)

    lanes = lax.iota(jnp.int32, 16)
    z16 = jnp.zeros((16,), jnp.int32)
    lane0 = lanes == 0

    pltpu.sync_copy(idx_hbm, idx_all)
    pltpu.sync_copy(tail_hbm, tail_v)

    def p1_body(v, off):
        iv = idx_all[pl.ds(v * 16, 16)]
        cu = jnp.minimum(iv, _TAIL - 1)
        c = cu >> _CSH
        m = (c >= base_c) & (c < base_c + n_c)
        mi = m.astype(jnp.int32)
        cnt = jnp.sum(mi)

        @pl.when(cnt > 0)
        def _():
            pos = jnp.minimum(off + plsc.cumsum(mi) - 1, BATCH - 1)
            plsc.store_scatter(pu, [pos], iv, mask=m)
            plsc.store_scatter(pg, [pos], v * 16 + lanes, mask=m)

        return off + cnt

    off = lax.fori_loop(0, BATCH // 16, p1_body, 0)

    def zinit(j, _):
        counts[pl.ds(j * 16, 16)] = z16
        return 0

    lax.fori_loop(0, 17, zinit, 0)

    def chunk_of(u):
        return (jnp.minimum(u, _TAIL - 1) >> _CSH) - base_c

    n_pv = (off + 15) >> 4

    def p2_body(v, _):
        uv = pu[pl.ds(v * 16, 16)]
        m = (v * 16 + lanes) < off
        c_l = jnp.clip(chunk_of(uv), 0, 255)
        rdc, lastm = plsc.scan_count(c_l, m)
        old = plsc.load_gather(counts, [c_l])
        plsc.store_scatter(counts, [c_l], old + rdc, mask=m & lastm)
        return 0

    lax.fori_loop(0, n_pv, p2_body, 0)

    def p3_body(j, run):
        cv = counts[pl.ds(j * 16, 16)]
        ex = plsc.cumsum(cv) - cv + run
        starts[pl.ds(j * 16, 16)] = ex
        cursor[pl.ds(j * 16, 16)] = ex
        return run + jnp.sum(cv)

    lax.fori_loop(0, 17, p3_body, 0)

    def p4_body(v, _):
        uv = pu[pl.ds(v * 16, 16)]
        gv = pg[pl.ds(v * 16, 16)]
        m = (v * 16 + lanes) < off
        c_l = jnp.clip(chunk_of(uv), 0, 255)
        rdc, lastm = plsc.scan_count(c_l, m)
        cur = plsc.load_gather(cursor, [c_l])
        pos = jnp.clip(cur + rdc - 1, 0, BATCH - 1)
        plsc.store_scatter(su, [pos], uv, mask=m)
        plsc.store_scatter(sg, [pos], gv, mask=m)
        plsc.store_scatter(cursor, [c_l], cur + rdc, mask=m & lastm)
        return 0

    lax.fori_loop(0, n_pv, p4_body, 0)

    scal[0] = 0

    def cnt_of(c):
        return counts[pl.ds(c, 16)][0]

    def hit(c):
        return (c < n_c) & (cnt_of(c) > 0)

    def issue(c, s):
        start = pl.multiple_of((base_c + c) * _CW, 128)
        pltpu.async_copy(tt_hbm.at[:, pl.ds(start, _CW)], bufs[s], gsems[s])

    def drain(s):
        pltpu.make_async_copy(
            tt_hbm.at[:, pl.ds(0, _CW)], bufs[s], gsems[s]
        ).wait()

    def extract_chunk(c, s):
        cbase = (base_c + c) * _CW
        s0 = starts[pl.ds(c, 16)][0]
        cnt = cnt_of(c)

        def ubody(p, _):
            u = su[pl.ds(p, 16)][0]
            g = sg[pl.ds(p, 16)][0]
            q = scal[0]
            scal[0] = q + 1
            slot = q & (_ORING - 1)
            obase = pl.multiple_of(slot * EMBED_DIM, EMBED_DIM)

            @pl.when(q >= _ORING)
            def _():
                pltpu.make_async_copy(
                    out_hbm.at[pl.ds(0, EMBED_DIM)],
                    oring.at[pl.ds(0, EMBED_DIM)],
                    osem,
                ).wait()

            lane = jnp.minimum(u - cbase, _CW - 1)
            is_tail = (u >= _TAIL).astype(jnp.int32)
            tail_m = (z16 + is_tail) != 0
            lane_v = z16 + lane
            tl_v = z16 + jnp.minimum(jnp.maximum(u - _TAIL, 0), EMBED_DIM - 1)
            for k in range(EMBED_DIM // 16):
                dvec = lanes + (k * 16)
                g1 = plsc.load_gather(bufs[s], [dvec, lane_v])
                g2 = plsc.load_gather(tail_v, [tl_v, dvec])
                oring[pl.ds(obase + k * 16, 16)] = jnp.where(tail_m, g2, g1)
            pltpu.async_copy(
                oring.at[pl.ds(obase, EMBED_DIM)],
                out_hbm.at[pl.ds(g * EMBED_DIM, EMBED_DIM)],
                osem,
            )
            return 0

        lax.fori_loop(s0, s0 + cnt, ubody, 0)

    for s in range(_NBUF):
        @pl.when(hit(s))
        def _(s=s):
            issue(s, s)

    def ring_body(t, _):
        for s in range(_NBUF):
            c = t * _NBUF + s

            @pl.when(hit(c))
            def _(c=c, s=s):
                drain(s)
                extract_chunk(c, s)

            @pl.when(hit(c + _NBUF))
            def _(c=c, s=s):
                issue(c + _NBUF, s)
        return 0

    lax.fori_loop(0, (_CPW + _XTRA + _NBUF - 1) // _NBUF, ring_body, 0)

    def odrain(r, _):
        pltpu.make_async_copy(
            out_hbm.at[pl.ds(0, EMBED_DIM)],
            oring.at[pl.ds(0, EMBED_DIM)],
            osem,
        ).wait()
        return 0

    lax.fori_loop(0, jnp.minimum(off, _ORING), odrain, 0)


def kernel(user_inputs, table):
    tt = jnp.transpose(table)
    tail = table[_TAIL:, :]
    out_flat = _sc_gather(user_inputs, tt, tail)
    return jnp.reshape(out_flat, (BATCH, EMBED_DIM))


# speedup vs baseline: 1.0895x; 1.0072x over previous
import functools

import jax
import jax.numpy as jnp
from jax import lax
from jax.experimental import pallas as pl
from jax.experimental.pallas import tpu as pltpu
from jax.experimental.pallas import tpu_sc as plsc

NUM_USERS = 1000000
EMBED_DIM = 64
BATCH = 16384

_info = plsc.get_sparse_core_info()
_NC = _info.num_cores
_NS = _info.num_subcores
_NW = _NC * _NS

_CW = 128
_TAIL = (NUM_USERS // 128) * 128
_NCH = _TAIL // _CW
_CPW = _NCH // _NW
_XTRA = _NCH - _CPW * _NW
_NBUF = 4
_ORING = 32
_CSH = 7


@functools.partial(
    pl.kernel,
    mesh=plsc.VectorSubcoreMesh(core_axis_name="c", subcore_axis_name="s"),
    out_type=jax.ShapeDtypeStruct((BATCH * EMBED_DIM,), jnp.float32),
    scratch_types=[
        pltpu.VMEM((BATCH,), jnp.int32),
        pltpu.VMEM((BATCH + 16,), jnp.int32),
        pltpu.VMEM((BATCH + 16,), jnp.int32),
        pltpu.VMEM((BATCH + 16,), jnp.int32),
        pltpu.VMEM((BATCH + 16,), jnp.int32),
        [pltpu.VMEM((EMBED_DIM, _CW), jnp.float32) for _ in range(_NBUF)],
        pltpu.VMEM((EMBED_DIM, EMBED_DIM), jnp.float32),
        pltpu.VMEM((_ORING * EMBED_DIM,), jnp.float32),
        pltpu.VMEM((272,), jnp.int32),
        pltpu.VMEM((272,), jnp.int32),
        pltpu.VMEM((272,), jnp.int32),
        pltpu.SMEM((8,), jnp.int32),
        [pltpu.SemaphoreType.DMA for _ in range(_NBUF)],
        pltpu.SemaphoreType.DMA,
    ],
    compiler_params=pltpu.CompilerParams(needs_layout_passes=False),
)
def _sc_gather(idx_hbm, tt_hbm, tail_hbm, out_hbm,
               idx_all, pu, pg, su, sg, bufs, tail_v, oring,
               counts, starts, cursor, scal,
               gsems, osem):
    wid = lax.axis_index("s") * _NC + lax.axis_index("c")
    base_c = wid * _CPW + jnp.minimum(wid, _XTRA)
    n_c = _CPW + (wid < _XTRA).astype(jnp.int32)

    lanes = lax.iota(jnp.int32, 16)
    z16 = jnp.zeros((16,), jnp.int32)
    lane0 = lanes == 0

    pltpu.sync_copy(idx_hbm, idx_all)
    pltpu.sync_copy(tail_hbm, tail_v)

    def p1_step(v, off):
        iv = idx_all[pl.ds(v * 16, 16)]
        cu = jnp.minimum(iv, _TAIL - 1)
        c = cu >> _CSH
        m = (c >= base_c) & (c < base_c + n_c)
        cnt = plsc.all_reduce_population_count(m)[0]

        @pl.when(cnt > 0)
        def _():
            mi = m.astype(jnp.int32)
            pos = jnp.minimum(off + plsc.cumsum(mi) - 1, BATCH - 1)
            plsc.store_scatter(pu, [pos], iv, mask=m)
            plsc.store_scatter(pg, [pos], v * 16 + lanes, mask=m)

        return off + cnt

    def p1_body(t, off):
        off = p1_step(2 * t, off)
        return p1_step(2 * t + 1, off)

    off = lax.fori_loop(0, BATCH // 32, p1_body, 0)

    def zinit(j, _):
        counts[pl.ds(j * 16, 16)] = z16
        return 0

    lax.fori_loop(0, 17, zinit, 0)

    def chunk_of(u):
        return (jnp.minimum(u, _TAIL - 1) >> _CSH) - base_c

    n_pv = (off + 15) >> 4

    def p2_body(v, _):
        uv = pu[pl.ds(v * 16, 16)]
        m = (v * 16 + lanes) < off
        c_l = jnp.clip(chunk_of(uv), 0, 255)
        rdc, lastm = plsc.scan_count(c_l, m)
        old = plsc.load_gather(counts, [c_l])
        plsc.store_scatter(counts, [c_l], old + rdc, mask=m & lastm)
        return 0

    lax.fori_loop(0, n_pv, p2_body, 0)

    def p3_body(j, run):
        cv = counts[pl.ds(j * 16, 16)]
        ex = plsc.cumsum(cv) - cv + run
        starts[pl.ds(j * 16, 16)] = ex
        cursor[pl.ds(j * 16, 16)] = ex
        return run + jnp.sum(cv)

    lax.fori_loop(0, 17, p3_body, 0)

    def p4_body(v, _):
        uv = pu[pl.ds(v * 16, 16)]
        gv = pg[pl.ds(v * 16, 16)]
        m = (v * 16 + lanes) < off
        c_l = jnp.clip(chunk_of(uv), 0, 255)
        rdc, lastm = plsc.scan_count(c_l, m)
        cur = plsc.load_gather(cursor, [c_l])
        pos = jnp.clip(cur + rdc - 1, 0, BATCH - 1)
        plsc.store_scatter(su, [pos], uv, mask=m)
        plsc.store_scatter(sg, [pos], gv, mask=m)
        plsc.store_scatter(cursor, [c_l], cur + rdc, mask=m & lastm)
        return 0

    lax.fori_loop(0, n_pv, p4_body, 0)

    scal[0] = 0

    def cnt_of(c):
        return counts[pl.ds(c, 16)][0]

    def hit(c):
        return (c < n_c) & (cnt_of(c) > 0)

    def issue(c, s):
        start = pl.multiple_of((base_c + c) * _CW, 128)
        pltpu.async_copy(tt_hbm.at[:, pl.ds(start, _CW)], bufs[s], gsems[s])

    def drain(s):
        pltpu.make_async_copy(
            tt_hbm.at[:, pl.ds(0, _CW)], bufs[s], gsems[s]
        ).wait()

    def extract_chunk(c, s):
        cbase = (base_c + c) * _CW
        s0 = starts[pl.ds(c, 16)][0]
        cnt = cnt_of(c)

        def ubody(p, _):
            u = su[pl.ds(p, 16)][0]
            g = sg[pl.ds(p, 16)][0]
            q = scal[0]
            scal[0] = q + 1
            slot = q & (_ORING - 1)
            obase = pl.multiple_of(slot * EMBED_DIM, EMBED_DIM)

            @pl.when(q >= _ORING)
            def _():
                pltpu.make_async_copy(
                    out_hbm.at[pl.ds(0, EMBED_DIM)],
                    oring.at[pl.ds(0, EMBED_DIM)],
                    osem,
                ).wait()

            lane = jnp.minimum(u - cbase, _CW - 1)
            is_tail = (u >= _TAIL).astype(jnp.int32)
            tail_m = (z16 + is_tail) != 0
            lane_v = z16 + lane
            tl_v = z16 + jnp.minimum(jnp.maximum(u - _TAIL, 0), EMBED_DIM - 1)
            for k in range(EMBED_DIM // 16):
                dvec = lanes + (k * 16)
                g1 = plsc.load_gather(bufs[s], [dvec, lane_v])
                g2 = plsc.load_gather(tail_v, [tl_v, dvec])
                oring[pl.ds(obase + k * 16, 16)] = jnp.where(tail_m, g2, g1)
            pltpu.async_copy(
                oring.at[pl.ds(obase, EMBED_DIM)],
                out_hbm.at[pl.ds(g * EMBED_DIM, EMBED_DIM)],
                osem,
            )
            return 0

        lax.fori_loop(s0, s0 + cnt, ubody, 0)

    for s in range(_NBUF):
        @pl.when(hit(s))
        def _(s=s):
            issue(s, s)

    def ring_body(t, _):
        for s in range(_NBUF):
            c = t * _NBUF + s

            @pl.when(hit(c))
            def _(c=c, s=s):
                drain(s)
                extract_chunk(c, s)

            @pl.when(hit(c + _NBUF))
            def _(c=c, s=s):
                issue(c + _NBUF, s)
        return 0

    lax.fori_loop(0, (_CPW + _XTRA + _NBUF - 1) // _NBUF, ring_body, 0)

    def odrain(r, _):
        pltpu.make_async_copy(
            out_hbm.at[pl.ds(0, EMBED_DIM)],
            oring.at[pl.ds(0, EMBED_DIM)],
            osem,
        ).wait()
        return 0

    lax.fori_loop(0, jnp.minimum(off, _ORING), odrain, 0)


def kernel(user_inputs, table):
    tt = jnp.transpose(table)
    tail = table[_TAIL:, :]
    out_flat = _sc_gather(user_inputs, tt, tail)
    return jnp.reshape(out_flat, (BATCH, EMBED_DIM))


# speedup vs baseline: 1.2018x; 1.1031x over previous
import functools

import jax
import jax.numpy as jnp
from jax import lax
from jax.experimental import pallas as pl
from jax.experimental.pallas import tpu as pltpu
from jax.experimental.pallas import tpu_sc as plsc

NUM_USERS = 1000000
EMBED_DIM = 64
BATCH = 16384

_info = plsc.get_sparse_core_info()
_NC = _info.num_cores
_NS = _info.num_subcores
_NW = _NC * _NS

_CW = 128
_TAIL = (NUM_USERS // 128) * 128
_NCH = _TAIL // _CW
_CPW = _NCH // _NW
_XTRA = _NCH - _CPW * _NW
_NBUF = 6
_ORING = 32
_CSH = 7


@functools.partial(
    pl.kernel,
    mesh=plsc.VectorSubcoreMesh(core_axis_name="c", subcore_axis_name="s"),
    out_type=jax.ShapeDtypeStruct((BATCH * EMBED_DIM,), jnp.float32),
    scratch_types=[
        pltpu.VMEM((BATCH + 16,), jnp.int32),
        pltpu.VMEM((BATCH + 16,), jnp.int32),
        pltpu.VMEM((BATCH + 16,), jnp.int32),
        pltpu.VMEM((BATCH + 16,), jnp.int32),
        [pltpu.VMEM((EMBED_DIM, _CW), jnp.float32) for _ in range(_NBUF)],
        pltpu.VMEM((EMBED_DIM, EMBED_DIM), jnp.float32),
        pltpu.VMEM((_ORING * EMBED_DIM,), jnp.float32),
        pltpu.VMEM((272,), jnp.int32),
        pltpu.VMEM((272,), jnp.int32),
        pltpu.VMEM((272,), jnp.int32),
        pltpu.SMEM((8,), jnp.int32),
        [pltpu.SemaphoreType.DMA for _ in range(_NBUF)],
        pltpu.SemaphoreType.DMA,
    ],
    compiler_params=pltpu.CompilerParams(needs_layout_passes=False),
)
def _sc_gather(idx_hbm, tt_hbm, tail_hbm, out_hbm,
               idx_all, pu, pg, sg, bufs, tail_v, oring,
               counts, starts, cursor, scal,
               gsems, osem):
    su = idx_all
    wid = lax.axis_index("s") * _NC + lax.axis_index("c")
    base_c = wid * _CPW + jnp.minimum(wid, _XTRA)
    n_c = _CPW + (wid < _XTRA).astype(jnp.int32)

    lanes = lax.iota(jnp.int32, 16)
    z16 = jnp.zeros((16,), jnp.int32)
    lane0 = lanes == 0

    pltpu.sync_copy(idx_hbm, idx_all.at[pl.ds(0, BATCH)])
    pltpu.sync_copy(tail_hbm, tail_v)

    def p1_step(v, off):
        iv = idx_all[pl.ds(v * 16, 16)]
        cu = jnp.minimum(iv, _TAIL - 1)
        c = cu >> _CSH
        m = (c >= base_c) & (c < base_c + n_c)
        cnt = plsc.all_reduce_population_count(m)[0]

        @pl.when(cnt > 0)
        def _():
            mi = m.astype(jnp.int32)
            pos = jnp.minimum(off + plsc.cumsum(mi) - 1, BATCH - 1)
            plsc.store_scatter(pu, [pos], iv, mask=m)
            plsc.store_scatter(pg, [pos], v * 16 + lanes, mask=m)

        return off + cnt

    def p1_body(t, off):
        off = p1_step(2 * t, off)
        return p1_step(2 * t + 1, off)

    off = lax.fori_loop(0, BATCH // 32, p1_body, 0)

    def zinit(j, _):
        counts[pl.ds(j * 16, 16)] = z16
        return 0

    lax.fori_loop(0, 17, zinit, 0)

    def chunk_of(u):
        return (jnp.minimum(u, _TAIL - 1) >> _CSH) - base_c

    n_pv = (off + 15) >> 4

    def p2_body(v, _):
        uv = pu[pl.ds(v * 16, 16)]
        m = (v * 16 + lanes) < off
        c_l = jnp.clip(chunk_of(uv), 0, 255)
        rdc, lastm = plsc.scan_count(c_l, m)
        old = plsc.load_gather(counts, [c_l])
        plsc.store_scatter(counts, [c_l], old + rdc, mask=m & lastm)
        return 0

    lax.fori_loop(0, n_pv, p2_body, 0)

    def p3_body(j, run):
        cv = counts[pl.ds(j * 16, 16)]
        ex = plsc.cumsum(cv) - cv + run
        starts[pl.ds(j * 16, 16)] = ex
        cursor[pl.ds(j * 16, 16)] = ex
        return run + jnp.sum(cv)

    lax.fori_loop(0, 17, p3_body, 0)

    def p4_body(v, _):
        uv = pu[pl.ds(v * 16, 16)]
        gv = pg[pl.ds(v * 16, 16)]
        m = (v * 16 + lanes) < off
        c_l = jnp.clip(chunk_of(uv), 0, 255)
        rdc, lastm = plsc.scan_count(c_l, m)
        cur = plsc.load_gather(cursor, [c_l])
        pos = jnp.clip(cur + rdc - 1, 0, BATCH - 1)
        plsc.store_scatter(su, [pos], uv, mask=m)
        plsc.store_scatter(sg, [pos], gv, mask=m)
        plsc.store_scatter(cursor, [c_l], cur + rdc, mask=m & lastm)
        return 0

    lax.fori_loop(0, n_pv, p4_body, 0)

    scal[0] = 0

    def cnt_of(c):
        return counts[pl.ds(c, 16)][0]

    def hit(c):
        return (c < n_c) & (cnt_of(c) > 0)

    def issue(c, s):
        start = pl.multiple_of((base_c + c) * _CW, 128)
        pltpu.async_copy(tt_hbm.at[:, pl.ds(start, _CW)], bufs[s], gsems[s])

    def drain(s):
        pltpu.make_async_copy(
            tt_hbm.at[:, pl.ds(0, _CW)], bufs[s], gsems[s]
        ).wait()

    def extract_chunk(c, s):
        cbase = (base_c + c) * _CW
        s0 = starts[pl.ds(c, 16)][0]
        cnt = cnt_of(c)

        def ubody(p, _):
            u = su[pl.ds(p, 16)][0]
            g = sg[pl.ds(p, 16)][0]
            q = scal[0]
            scal[0] = q + 1
            slot = q & (_ORING - 1)
            obase = pl.multiple_of(slot * EMBED_DIM, EMBED_DIM)

            @pl.when(q >= _ORING)
            def _():
                pltpu.make_async_copy(
                    out_hbm.at[pl.ds(0, EMBED_DIM)],
                    oring.at[pl.ds(0, EMBED_DIM)],
                    osem,
                ).wait()

            lane = jnp.minimum(u - cbase, _CW - 1)
            is_tail = (u >= _TAIL).astype(jnp.int32)
            tail_m = (z16 + is_tail) != 0
            lane_v = z16 + lane
            tl_v = z16 + jnp.minimum(jnp.maximum(u - _TAIL, 0), EMBED_DIM - 1)
            for k in range(EMBED_DIM // 16):
                dvec = lanes + (k * 16)
                g1 = plsc.load_gather(bufs[s], [dvec, lane_v])
                g2 = plsc.load_gather(tail_v, [tl_v, dvec])
                oring[pl.ds(obase + k * 16, 16)] = jnp.where(tail_m, g2, g1)
            pltpu.async_copy(
                oring.at[pl.ds(obase, EMBED_DIM)],
                out_hbm.at[pl.ds(g * EMBED_DIM, EMBED_DIM)],
                osem,
            )
            return 0

        lax.fori_loop(s0, s0 + cnt, ubody, 0)

    for s in range(_NBUF):
        @pl.when(hit(s))
        def _(s=s):
            issue(s, s)

    def ring_body(t, _):
        for s in range(_NBUF):
            c = t * _NBUF + s

            @pl.when(hit(c))
            def _(c=c, s=s):
                drain(s)
                extract_chunk(c, s)

            @pl.when(hit(c + _NBUF))
            def _(c=c, s=s):
                issue(c + _NBUF, s)
        return 0

    lax.fori_loop(0, (_CPW + _XTRA + _NBUF - 1) // _NBUF, ring_body, 0)

    def odrain(r, _):
        pltpu.make_async_copy(
            out_hbm.at[pl.ds(0, EMBED_DIM)],
            oring.at[pl.ds(0, EMBED_DIM)],
            osem,
        ).wait()
        return 0

    lax.fori_loop(0, jnp.minimum(off, _ORING), odrain, 0)


def kernel(user_inputs, table):
    tt = jnp.transpose(table)
    tail = table[_TAIL:, :]
    out_flat = _sc_gather(user_inputs, tt, tail)
    return jnp.reshape(out_flat, (BATCH, EMBED_DIM))
